# baseline (speedup 1.0000x reference)
import functools

import jax
import jax.numpy as jnp
from jax import lax
from jax.experimental import pallas as pl
from jax.experimental.pallas import tpu as pltpu
from jax.experimental.pallas import tpu_sc as plsc

F32 = jnp.float32
I32 = jnp.int32

NTILES = 32


def _mesh():
    return plsc.VectorSubcoreMesh(core_axis_name="c", subcore_axis_name="s")


def _lane_gather(x, idx):
    dnums = lax.GatherDimensionNumbers(
        offset_dims=(), collapsed_slice_dims=(0,), start_index_map=(0,))
    return lax.gather(x, idx[:, None], dnums, (1,),
                      mode=lax.GatherScatterMode.PROMISE_IN_BOUNDS)


def _sc_segment_sum(D, per_tile, SB, acc_rows, table_rows):
    nsub = per_tile // SB
    assert nsub % 2 == 0
    rows_pt = acc_rows // 16

    @functools.partial(
        pl.kernel,
        mesh=_mesh(),
        compiler_params=pltpu.CompilerParams(use_tc_tiling_on_sc=False),
        out_type=jax.ShapeDtypeStruct((2, acc_rows, D), F32),
        scratch_types=[
            pltpu.VMEM_SHARED((acc_rows, D), F32),
            pltpu.VMEM((nsub, SB), I32),
            pltpu.VMEM((per_tile,), I32),
            pltpu.VMEM((2, SB, D), F32),
            pltpu.SemaphoreType.DMA,
            pltpu.SemaphoreType.DMA,
        ],
    )
    def k(src_hbm, dst_hbm, tab_hbm, zeros_hbm, out_hbm, acc, dsti, srci, rows,
          sem0, sem1):
        c = lax.axis_index("c")
        s = lax.axis_index("s")
        wid = c * 16 + s

        pltpu.sync_copy(
            zeros_hbm.at[pl.ds(s * rows_pt, rows_pt), :],
            acc.at[pl.ds(s * rows_pt, rows_pt), :])
        pltpu.sync_copy(src_hbm.at[pl.ds(wid * per_tile, per_tile)], srci)
        pltpu.sync_copy(dst_hbm.at[wid], dsti)
        plsc.subcore_barrier()

        pltpu.async_copy(tab_hbm.at[srci.at[pl.ds(0, SB)]], rows.at[0], sem0)

        def step(i, carry):
            j = i * 2
            pltpu.async_copy(
                tab_hbm.at[srci.at[pl.ds((j + 1) * SB, SB)]], rows.at[1], sem1)
            pltpu.make_async_copy(
                tab_hbm.at[srci.at[pl.ds(j * SB, SB)]], rows.at[0], sem0).wait()
            pltpu.sync_copy(rows.at[0], acc.at[dsti.at[j]], add=True)

            @pl.when(j + 2 < nsub)
            def _():
                pltpu.async_copy(
                    tab_hbm.at[srci.at[pl.ds((j + 2) * SB, SB)]], rows.at[0], sem0)

            pltpu.make_async_copy(
                tab_hbm.at[srci.at[pl.ds((j + 1) * SB, SB)]], rows.at[1], sem1).wait()
            pltpu.sync_copy(rows.at[1], acc.at[dsti.at[j + 1]], add=True)
            return carry

        lax.fori_loop(0, nsub // 2, step, 0)
        plsc.subcore_barrier()
        pltpu.sync_copy(
            acc.at[pl.ds(s * rows_pt, rows_pt), :],
            out_hbm.at[c, pl.ds(s * rows_pt, rows_pt), :],
        )

    return k


def _sc_segment_max(n_entries, chunk, acc_rows, out_cols, n_pts_pad):
    nchunks = n_entries // chunk
    cpts = chunk // 4
    nbatch = chunk // 16

    @functools.partial(
        pl.kernel,
        mesh=_mesh(),
        compiler_params=pltpu.CompilerParams(
            needs_layout_passes=False, use_tc_tiling_on_sc=False),
        out_type=jax.ShapeDtypeStruct((NTILES * out_cols,), F32),
        scratch_types=[
            pltpu.VMEM((acc_rows,), F32),
            pltpu.VMEM((chunk,), I32),
            pltpu.VMEM((cpts,), F32),
        ],
    )
    def k(splat_hbm, ht_hbm, out_hbm, acc, idxb, hb):
        c = lax.axis_index("c")
        s = lax.axis_index("s")
        wid = c * 16 + s
        hbase = wid * n_pts_pad
        obase = wid * out_cols
        lanes = lax.iota(I32, 16)
        neg = jnp.full((16,), -jnp.inf, F32)

        def init(i, carry):
            acc[pl.ds(i * 16, 16)] = neg
            return carry

        lax.fori_loop(0, acc_rows // 16, init, 0)

        def do_chunk(ci, carry):
            pltpu.sync_copy(splat_hbm.at[pl.ds(ci * chunk, chunk)], idxb)
            pltpu.sync_copy(ht_hbm.at[pl.ds(hbase + ci * cpts, cpts)], hb)

            def batch(b, bc):
                kv = idxb[pl.ds(b * 16, 16)]
                pidx = lax.shift_right_logical(b * 16 + lanes, 2)
                vals = plsc.load_gather(hb, [pidx])
                ks, vs = plsc.sort_key_val(kv, vals)
                for sh in (1, 2, 4, 8):
                    sl = jnp.maximum(lanes - sh, 0)
                    k2 = _lane_gather(ks, sl)
                    v2 = _lane_gather(vs, sl)
                    m = (k2 == ks) & (lanes >= sh)
                    vs = jnp.where(m, jnp.maximum(vs, v2), vs)
                nxt = _lane_gather(ks, jnp.minimum(lanes + 1, 15))
                is_last = (ks != nxt) | (lanes == 15)
                cur = plsc.load_gather(acc, [ks])
                plsc.store_scatter(acc, [ks], jnp.maximum(cur, vs), mask=is_last)
                return bc

            lax.fori_loop(0, nbatch, batch, 0)
            return carry

        lax.fori_loop(0, nchunks, do_chunk, 0)

        def fin(i, carry):
            t = acc[pl.ds(i * 16, 16)]
            acc[pl.ds(i * 16, 16)] = jnp.where(t == neg, jnp.zeros((16,), F32), t)
            return carry

        lax.fori_loop(0, out_cols // 16, fin, 0)
        pltpu.sync_copy(acc.at[pl.ds(0, out_cols)], out_hbm.at[pl.ds(obase, out_cols)])

    return k


def _sc_gather_rows(D, per_tile, SB, table_rows):
    nsub = per_tile // SB
    assert nsub % 2 == 0
    n_out = NTILES * per_tile

    @functools.partial(
        pl.kernel,
        mesh=_mesh(),
        compiler_params=pltpu.CompilerParams(use_tc_tiling_on_sc=False),
        out_type=jax.ShapeDtypeStruct((n_out, D), F32),
        scratch_types=[
            pltpu.VMEM((per_tile,), I32),
            pltpu.VMEM((2, SB, D), F32),
            pltpu.SemaphoreType.DMA,
            pltpu.SemaphoreType.DMA,
        ],
    )
    def k(idx_hbm, tab_hbm, out_hbm, idxb, rows, sem0, sem1):
        c = lax.axis_index("c")
        s = lax.axis_index("s")
        wid = c * 16 + s
        base = wid * per_tile
        pltpu.sync_copy(idx_hbm.at[pl.ds(base, per_tile)], idxb)

        pltpu.async_copy(tab_hbm.at[idxb.at[pl.ds(0, SB)]], rows.at[0], sem0)

        def step(i, carry):
            j = i * 2
            pltpu.async_copy(
                tab_hbm.at[idxb.at[pl.ds((j + 1) * SB, SB)]], rows.at[1], sem1)
            pltpu.make_async_copy(
                tab_hbm.at[idxb.at[pl.ds(j * SB, SB)]], rows.at[0], sem0).wait()
            pltpu.sync_copy(rows.at[0], out_hbm.at[pl.ds(base + j * SB, SB), :])

            @pl.when(j + 2 < nsub)
            def _():
                pltpu.async_copy(
                    tab_hbm.at[idxb.at[pl.ds((j + 2) * SB, SB)]], rows.at[0], sem0)

            pltpu.make_async_copy(
                tab_hbm.at[idxb.at[pl.ds((j + 1) * SB, SB)]], rows.at[1], sem1).wait()
            pltpu.sync_copy(
                rows.at[1], out_hbm.at[pl.ds(base + (j + 1) * SB, SB), :])
            return carry

        lax.fori_loop(0, nsub // 2, step, 0)

    return k


def _sc_gather_sum4(D, per_tile, SB, table_rows):
    nsub = per_tile // SB
    n_out = NTILES * per_tile

    assert nsub % 2 == 0

    @functools.partial(
        pl.kernel,
        mesh=_mesh(),
        compiler_params=pltpu.CompilerParams(use_tc_tiling_on_sc=False),
        out_type=jax.ShapeDtypeStruct((n_out, D), F32),
        scratch_types=[
            pltpu.VMEM((4, per_tile), I32),
            pltpu.VMEM((8, SB, D), F32),
            pltpu.VMEM((SB, D), F32),
            pltpu.SemaphoreType.DMA,
            pltpu.SemaphoreType.DMA,
        ],
    )
    def k(idx_hbm, tab_hbm, out_hbm, idxb, rows, obuf, sem0, sem1):
        c = lax.axis_index("c")
        s = lax.axis_index("s")
        wid = c * 16 + s
        base = wid * per_tile
        for jj in range(4):
            pltpu.sync_copy(idx_hbm.at[jj, pl.ds(base, per_tile)], idxb.at[jj])

        def fire(j, g, sem):
            for jj in range(4):
                pltpu.async_copy(
                    tab_hbm.at[idxb.at[jj, pl.ds(j * SB, SB)]],
                    rows.at[g * 4 + jj], sem)

        def drain_sum_out(j, g, sem):
            for jj in range(4):
                pltpu.make_async_copy(
                    tab_hbm.at[idxb.at[jj, pl.ds(j * SB, SB)]],
                    rows.at[g * 4 + jj], sem).wait()

            def row(i, carry):
                for t in range(D // 16):
                    sl = pl.ds(t * 16, 16)
                    v = rows[g * 4 + 0, i, sl] + rows[g * 4 + 1, i, sl]
                    w = rows[g * 4 + 2, i, sl] + rows[g * 4 + 3, i, sl]
                    obuf[i, sl] = v + w
                return carry

            lax.fori_loop(0, SB, row, 0)
            pltpu.sync_copy(obuf, out_hbm.at[pl.ds(base + j * SB, SB), :])

        fire(0, 0, sem0)

        def pair(i, carry):
            j = i * 2
            fire(j + 1, 1, sem1)
            drain_sum_out(j, 0, sem0)

            @pl.when(j + 2 < nsub)
            def _():
                fire(j + 2, 0, sem0)

            drain_sum_out(j + 1, 1, sem1)
            return carry

        lax.fori_loop(0, nsub // 2, pair, 0)

    return k


def _rb(r, c):
    return pl.BlockSpec((r, c), lambda i: (i, 0))


def _rb3(d0, r, c):
    return pl.BlockSpec((d0, r, c), lambda i: (0, i, 0))


def _full(shape):
    nd = len(shape)
    return pl.BlockSpec(shape, lambda i: (0,) * nd)


def _dot(a, b):
    return jnp.dot(a, b, preferred_element_type=F32)


def _tc_pointnet(featT, w1t, b1, w2t, b2, w3t, b3, np_pad, bs=1024):
    def body(f, wa, ba, wb, bb, wc, bc, o):
        h = jnp.maximum(_dot(wa[...], f[...]) + ba[...], 0.0)
        h = jnp.maximum(_dot(wb[...], h) + bb[...], 0.0)
        o[...] = _dot(wc[...], h) + bc[...]

    return pl.pallas_call(
        body,
        grid=(np_pad // bs,),
        in_specs=[
            pl.BlockSpec((4, bs), lambda i: (0, i)),
            _full((8, 4)), _full((8, 1)),
            _full((16, 8)), _full((16, 1)),
            _full((32, 16)), _full((32, 1)),
        ],
        out_specs=pl.BlockSpec((32, bs), lambda i: (0, i)),
        out_shape=jax.ShapeDtypeStruct((32, np_pad), F32),
    )(featT, w1t, b1, w2t, b2, w3t, b3)


def _tc_invdeg_col(agg, col, aw, n_rows, bs):
    def body(d, o):
        t = d[0][:, col:col + 1] + d[1][:, col:col + 1]
        r = 1.0 / jnp.maximum(t, 1.0)
        o[...] = jnp.concatenate([r] * 16, axis=1)

    return pl.pallas_call(
        body,
        grid=(n_rows // bs,),
        in_specs=[_rb3(2, bs, aw)],
        out_specs=_rb(bs, 16),
        out_shape=jax.ShapeDtypeStruct((n_rows, 16), F32),
    )(agg)


def _aug_ones(u):
    r = u.shape[0]
    return jnp.concatenate(
        [u, jnp.ones((r, 1), F32), jnp.zeros((r, 15), F32)], axis=1)


def _tc_stage1(lv_raw, ws, wn, b, n_rows, bs=512):
    def body(x, a, c, bb, x0, so, uo):
        xv = x[...]
        xr = jnp.maximum(xv, 0.0)
        x0[...] = xv
        so[...] = _dot(xr, a[...]) + bb[...]
        uo[...] = _aug_ones(_dot(xr, c[...]))

    return pl.pallas_call(
        body,
        grid=(n_rows // bs,),
        in_specs=[_rb(bs, 32), _full((32, 32)), _full((32, 32)), _full((1, 32))],
        out_specs=[_rb(bs, 32), _rb(bs, 32), _rb(bs, 48)],
        out_shape=[
            jax.ShapeDtypeStruct((n_rows, 32), F32),
            jax.ShapeDtypeStruct((n_rows, 32), F32),
            jax.ShapeDtypeStruct((n_rows, 48), F32),
        ],
    )(lv_raw, ws, wn, b)


def _tc_combine_preconv(s_prev, agg2, invd, x_prev, ws, wn, b, n_rows, cx,
                        bs=512, aw=32):
    def body(sp, ag, iv, xp, a, c, bb, yo, so, uo):
        y = sp[...] + (ag[0][:, :32] + ag[1][:, :32]) * iv[..., :1]
        z = jnp.concatenate([xp[...], y], axis=1)
        zr = jnp.maximum(z, 0.0)
        yo[...] = y
        so[...] = _dot(zr, a[...]) + bb[...]
        uo[...] = _dot(zr, c[...])

    return pl.pallas_call(
        body,
        grid=(n_rows // bs,),
        in_specs=[
            _rb(bs, 32), _rb3(2, bs, aw), _rb(bs, 16), _rb(bs, cx),
            _full((cx + 32, 32)), _full((cx + 32, 32)), _full((1, 32)),
        ],
        out_specs=[_rb(bs, 32), _rb(bs, 32), _rb(bs, 32)],
        out_shape=[
            jax.ShapeDtypeStruct((n_rows, 32), F32),
            jax.ShapeDtypeStruct((n_rows, 32), F32),
            jax.ShapeDtypeStruct((n_rows, 32), F32),
        ],
    )(s_prev, agg2, invd, x_prev, ws, wn, b)


def _tc_finish_block(s_prev, agg2, invd, x_prev, y1, n_rows, cx, relu_out, bs=512):
    def body(sp, ag, iv, xp, y1r, lvo, xro):
        y2 = sp[...] + (ag[0] + ag[1]) * iv[..., :1]
        lv = jnp.concatenate([xp[...], y1r[...], y2], axis=1)
        lvo[...] = lv
        xro[...] = _aug_ones(jnp.maximum(lv, 0.0))

    def body_norelu(sp, ag, iv, xp, y1r, lvo):
        y2 = sp[...] + (ag[0] + ag[1]) * iv[..., :1]
        lvo[...] = jnp.concatenate([xp[...], y1r[...], y2], axis=1)

    cout = cx + 64
    outs = [jax.ShapeDtypeStruct((n_rows, cout), F32)]
    out_specs = [_rb(bs, cout)]
    if relu_out:
        outs.append(jax.ShapeDtypeStruct((n_rows, cout + 16), F32))
        out_specs.append(_rb(bs, cout + 16))
    return pl.pallas_call(
        body if relu_out else body_norelu,
        grid=(n_rows // bs,),
        in_specs=[_rb(bs, 32), _rb3(2, bs, 32), _rb(bs, 16), _rb(bs, cx), _rb(bs, 32)],
        out_specs=out_specs,
        out_shape=outs,
    )(s_prev, agg2, invd, x_prev, y1)


def _tc_down(coar2, wdown, bdown, wsa, wna, ba, n_rows, bs=448):
    def body(cg, wd, bd, a, c, bb, lvo, so, uo):
        csum = cg[0] + cg[1]
        iv = 1.0 / jnp.maximum(csum[:, 96:97], 1.0)
        lv2 = _dot(csum[:, :96] * iv, wd[...]) + bd[...]
        xr = jnp.maximum(lv2, 0.0)
        lvo[...] = lv2
        so[...] = _dot(xr, a[...]) + bb[...]
        uo[...] = _aug_ones(_dot(xr, c[...]))

    return pl.pallas_call(
        body,
        grid=(n_rows // bs,),
        in_specs=[
            _rb3(2, bs, 112), _full((96, 96)), _full((1, 96)),
            _full((96, 32)), _full((96, 32)), _full((1, 32)),
        ],
        out_specs=[_rb(bs, 96), _rb(bs, 32), _rb(bs, 48)],
        out_shape=[
            jax.ShapeDtypeStruct((n_rows, 96), F32),
            jax.ShapeDtypeStruct((n_rows, 32), F32),
            jax.ShapeDtypeStruct((n_rows, 48), F32),
        ],
    )(coar2, wdown, bdown, wsa, wna, ba)


def _tc_up(s_prev, agg2, invd, lv2, y3, wup, bup, n_rows, bs=448):
    def body(sp, ag, iv, l2, y3r, wu, bu, upo):
        y4 = sp[...] + (ag[0] + ag[1]) * iv[..., :1]
        lvf = jnp.concatenate([l2[...], y3r[...], y4], axis=1)
        upo[...] = _dot(jnp.maximum(lvf, 0.0), wu[...]) + bu[...]

    return pl.pallas_call(
        body,
        grid=(n_rows // bs,),
        in_specs=[
            _rb(bs, 32), _rb3(2, bs, 32), _rb(bs, 16), _rb(bs, 96), _rb(bs, 32),
            _full((160, 96)), _full((1, 96)),
        ],
        out_specs=_rb(bs, 96),
        out_shape=jax.ShapeDtypeStruct((n_rows, 96), F32),
    )(s_prev, agg2, invd, lv2, y3, wup, bup)


def _tc_skip_preconv(upg, lv96, wsa, wna, ba, n_rows, bs=512):
    def body(ug, l9, a, c, bb, lvo, so, uo):
        lv = ug[...] + l9[...]
        xr = jnp.maximum(lv, 0.0)
        lvo[...] = lv
        so[...] = _dot(xr, a[...]) + bb[...]
        uo[...] = _dot(xr, c[...])

    return pl.pallas_call(
        body,
        grid=(n_rows // bs,),
        in_specs=[
            _rb(bs, 96), _rb(bs, 96),
            _full((96, 32)), _full((96, 32)), _full((1, 32)),
        ],
        out_specs=[_rb(bs, 96), _rb(bs, 32), _rb(bs, 32)],
        out_shape=[
            jax.ShapeDtypeStruct((n_rows, 96), F32),
            jax.ShapeDtypeStruct((n_rows, 32), F32),
            jax.ShapeDtypeStruct((n_rows, 32), F32),
        ],
    )(upg, lv96, wsa, wna, ba)


def _tc_head(gsum, wsl, bsl, w1, b1, w2, b2, n_rows, bs=512):
    def body(gh, a, ba, bmat, bb, cmat, bc, o):
        g = gh[...] * 0.25
        sv = _dot(jnp.maximum(g, 0.0), a[...]) + ba[...]
        h2 = jnp.maximum(_dot(jnp.maximum(sv, 0.0), bmat[...]) + bb[...], 0.0)
        lg = _dot(h2, cmat[...]) + bc[...]
        mx = jnp.max(lg, axis=1, keepdims=True)
        e = jnp.exp(lg - mx)
        o[...] = lg - mx - jnp.log(jnp.sum(e, axis=1, keepdims=True))

    return pl.pallas_call(
        body,
        grid=(n_rows // bs,),
        in_specs=[
            _rb(bs, 160),
            _full((160, 64)), _full((1, 64)),
            _full((64, 32)), _full((1, 32)),
            _full((32, 20)), _full((1, 20)),
        ],
        out_specs=_rb(bs, 20),
        out_shape=jax.ShapeDtypeStruct((n_rows, 20), F32),
    )(gsum, wsl, bsl, w1, b1, w2, b2)


def kernel(positions, values, params, splat_idx, edge_index, coarse_idx, coarse_edge_index):
    p = params
    n = positions.shape[0]
    v = 25000
    e = edge_index.shape[1]
    ec = coarse_edge_index.shape[1]

    np_pad = 51200
    vp = 25088
    vacc = 25216
    vdum = 25100
    vcp = 6272
    vcacc = 6400
    vcdum = 6300

    i32 = lambda x: x.astype(I32)

    ent = 4 * n
    ent_pad = 204800
    splat_flat = i32(splat_idx.reshape(-1))
    splat_max = jnp.concatenate(
        [splat_flat, jnp.full((ent_pad - ent,), vdum, I32)])
    splat_T = jnp.concatenate(
        [i32(splat_idx), jnp.zeros((np_pad - n, 4), I32)]).T

    e_pad = 409600
    src_f = jnp.concatenate([i32(edge_index[0]), jnp.zeros((e_pad - e,), I32)])
    dst_f = jnp.concatenate(
        [i32(edge_index[1]), jnp.full((e_pad - e,), vdum, I32)]
    ).reshape(NTILES, 100, 128)

    ec_pad = 102400
    src_c = jnp.concatenate([i32(coarse_edge_index[0]), jnp.zeros((ec_pad - ec,), I32)])
    dst_c = jnp.concatenate(
        [i32(coarse_edge_index[1]), jnp.full((ec_pad - ec,), vcdum, I32)]
    ).reshape(NTILES, 20, 160)

    v_pad = 25600
    src_s = jnp.concatenate([jnp.arange(v, dtype=I32), jnp.zeros((v_pad - v,), I32)])
    dst_s = jnp.concatenate(
        [i32(coarse_idx), jnp.full((v_pad - v,), vcdum, I32)]
    ).reshape(NTILES, 10, 80)
    cidx_gather = jnp.concatenate([i32(coarse_idx), jnp.zeros((v_pad - v,), I32)])

    seg48_fine = _sc_segment_sum(48, 12800, 128, vacc, vp)
    seg32_fine = _sc_segment_sum(32, 12800, 128, vacc, vp)
    seg48_coarse = _sc_segment_sum(48, 3200, 160, vcacc, vcp)
    seg32_coarse = _sc_segment_sum(32, 3200, 160, vcacc, vcp)
    seg112_down = _sc_segment_sum(112, 800, 80, vcacc, vp)
    segmax_k = _sc_segment_max(ent_pad, 1600, vacc, vp, np_pad)
    gather96 = _sc_gather_rows(96, 800, 80, vcp)
    gathersum160 = _sc_gather_sum4(160, 1600, 40, vp)

    zf48 = jnp.zeros((vacc, 48), F32)
    zf32 = jnp.zeros((vacc, 32), F32)
    zc48 = jnp.zeros((vcacc, 48), F32)
    zc32 = jnp.zeros((vcacc, 32), F32)
    zc112 = jnp.zeros((vcacc, 112), F32)

    feat = jnp.concatenate([positions, values], axis=1)
    featT = jnp.concatenate(
        [feat, jnp.zeros((np_pad - n, 4), F32)]).T
    hT = _tc_pointnet(
        featT,
        p['W_p1'].T, p['b_p1'].reshape(8, 1),
        p['W_p2'].T, p['b_p2'].reshape(16, 1),
        p['W_p3'].T, p['b_p3'].reshape(32, 1),
        np_pad)
    lv0 = segmax_k(splat_max, hT.reshape(-1)).reshape(NTILES, vp).T

    x0, s1, u1 = _tc_stage1(lv0, p['d1a_s'], p['d1a_n'], p['d1a_b'].reshape(1, 32), vp)
    agg1 = seg48_fine(src_f, dst_f, u1, zf48)
    invd_f = _tc_invdeg_col(agg1, 32, 48, vp, 512)
    y1, s2, u2 = _tc_combine_preconv(
        s1, agg1, invd_f, x0, p['d1b_s'], p['d1b_n'], p['d1b_b'].reshape(1, 32),
        vp, 32, aw=48)
    agg2 = seg32_fine(src_f, dst_f, u2, zf32)
    lv96, xr112 = _tc_finish_block(s2, agg2, invd_f, x0, y1, vp, 32, True)

    coar = seg112_down(src_s, dst_s, xr112, zc112)
    lv2, s3, u3 = _tc_down(
        coar, p['W_down'], p['b_down'].reshape(1, 96),
        p['d2a_s'], p['d2a_n'], p['d2a_b'].reshape(1, 32), vcp)
    agg3 = seg48_coarse(src_c, dst_c, u3, zc48)
    invd_c = _tc_invdeg_col(agg3, 32, 48, vcp, 448)
    y3, s4, u4 = _tc_combine_preconv(
        s3, agg3, invd_c, lv2, p['d2b_s'], p['d2b_n'], p['d2b_b'].reshape(1, 32),
        vcp, 96, bs=448, aw=48)
    agg4 = seg32_coarse(src_c, dst_c, u4, zc32)
    up = _tc_up(s4, agg4, invd_c, lv2, y3, p['W_up'], p['b_up'].reshape(1, 96), vcp)

    upg = gather96(cidx_gather, up)
    lv96b, s5, u5 = _tc_skip_preconv(
        upg[:vp], lv96, p['d3a_s'], p['d3a_n'], p['d3a_b'].reshape(1, 32), vp)
    agg5 = seg32_fine(src_f, dst_f, u5, zf32)
    y5, s6, u6 = _tc_combine_preconv(
        s5, agg5, invd_f, lv96b, p['d3b_s'], p['d3b_n'], p['d3b_b'].reshape(1, 32), vp, 96)
    agg6 = seg32_fine(src_f, dst_f, u6, zf32)
    (lvfin,) = _tc_finish_block(s6, agg6, invd_f, lv96b, y5, vp, 96, False)

    gsum = gathersum160(splat_T, lvfin)
    out = _tc_head(
        gsum, p['W_sl'], p['b_sl'].reshape(1, 64),
        p['W_s1'], p['b_s1'].reshape(1, 32),
        p['W_s2'], p['b_s2'].reshape(1, 20), np_pad)
    return out[:n].reshape(1, n, 20)


# speedup vs baseline: 1.0944x; 1.0944x over previous
import functools

import jax
import jax.numpy as jnp
from jax import lax
from jax.experimental import pallas as pl
from jax.experimental.pallas import tpu as pltpu
from jax.experimental.pallas import tpu_sc as plsc

F32 = jnp.float32
I32 = jnp.int32

NTILES = 32


def _mesh():
    return plsc.VectorSubcoreMesh(core_axis_name="c", subcore_axis_name="s")


def _lane_gather(x, idx):
    dnums = lax.GatherDimensionNumbers(
        offset_dims=(), collapsed_slice_dims=(0,), start_index_map=(0,))
    return lax.gather(x, idx[:, None], dnums, (1,),
                      mode=lax.GatherScatterMode.PROMISE_IN_BOUNDS)


def _sc_segment_sum(D, per_tile, SB, acc_rows, table_rows):
    nsub = per_tile // SB
    assert nsub % 2 == 0
    rows_pt = acc_rows // 16

    @functools.partial(
        pl.kernel,
        mesh=_mesh(),
        compiler_params=pltpu.CompilerParams(use_tc_tiling_on_sc=False),
        out_type=jax.ShapeDtypeStruct((2, acc_rows, D), F32),
        scratch_types=[
            pltpu.VMEM_SHARED((acc_rows, D), F32),
            pltpu.VMEM((nsub, SB), I32),
            pltpu.VMEM((per_tile,), I32),
            pltpu.VMEM((2, SB, D), F32),
            pltpu.SemaphoreType.DMA,
            pltpu.SemaphoreType.DMA,
        ],
    )
    def k(src_hbm, dst_hbm, tab_hbm, zeros_hbm, out_hbm, acc, dsti, srci, rows,
          sem0, sem1):
        c = lax.axis_index("c")
        s = lax.axis_index("s")
        wid = c * 16 + s

        pltpu.sync_copy(
            zeros_hbm.at[pl.ds(s * rows_pt, rows_pt), :],
            acc.at[pl.ds(s * rows_pt, rows_pt), :])
        pltpu.sync_copy(src_hbm.at[pl.ds(wid * per_tile, per_tile)], srci)
        pltpu.sync_copy(dst_hbm.at[wid], dsti)
        plsc.subcore_barrier()

        pltpu.async_copy(tab_hbm.at[srci.at[pl.ds(0, SB)]], rows.at[0], sem0)

        def step(i, carry):
            j = i * 2
            pltpu.async_copy(
                tab_hbm.at[srci.at[pl.ds((j + 1) * SB, SB)]], rows.at[1], sem1)
            pltpu.make_async_copy(
                tab_hbm.at[srci.at[pl.ds(j * SB, SB)]], rows.at[0], sem0).wait()
            pltpu.sync_copy(rows.at[0], acc.at[dsti.at[j]], add=True)

            @pl.when(j + 2 < nsub)
            def _():
                pltpu.async_copy(
                    tab_hbm.at[srci.at[pl.ds((j + 2) * SB, SB)]], rows.at[0], sem0)

            pltpu.make_async_copy(
                tab_hbm.at[srci.at[pl.ds((j + 1) * SB, SB)]], rows.at[1], sem1).wait()
            pltpu.sync_copy(rows.at[1], acc.at[dsti.at[j + 1]], add=True)
            return carry

        lax.fori_loop(0, nsub // 2, step, 0)
        plsc.subcore_barrier()
        pltpu.sync_copy(
            acc.at[pl.ds(s * rows_pt, rows_pt), :],
            out_hbm.at[c, pl.ds(s * rows_pt, rows_pt), :],
        )

    return k


def _sc_degree(per_tile, SB, acc_rows):
    nsub = per_tile // SB
    rows_pt = acc_rows // 16
    D = 8

    @functools.partial(
        pl.kernel,
        mesh=_mesh(),
        compiler_params=pltpu.CompilerParams(use_tc_tiling_on_sc=False),
        out_type=jax.ShapeDtypeStruct((2, acc_rows, D), F32),
        scratch_types=[
            pltpu.VMEM_SHARED((acc_rows, D), F32),
            pltpu.VMEM((nsub, SB), I32),
            pltpu.VMEM((SB, D), F32),
            pltpu.VMEM((rows_pt, D), F32),
        ],
    )
    def k(dst_hbm, ones_hbm, zeros_hbm, out_hbm, acc, dsti, ones, zbuf):
        c = lax.axis_index("c")
        s = lax.axis_index("s")
        wid = c * 16 + s

        pltpu.sync_copy(zeros_hbm, zbuf)
        pltpu.sync_copy(ones_hbm, ones)
        pltpu.sync_copy(zbuf, acc.at[pl.ds(s * rows_pt, rows_pt), :])
        pltpu.sync_copy(dst_hbm.at[wid], dsti)
        plsc.subcore_barrier()

        def step(j, carry):
            pltpu.sync_copy(ones, acc.at[dsti.at[j]], add=True)
            return carry

        lax.fori_loop(0, nsub, step, 0)
        plsc.subcore_barrier()
        pltpu.sync_copy(
            acc.at[pl.ds(s * rows_pt, rows_pt), :],
            out_hbm.at[c, pl.ds(s * rows_pt, rows_pt), :],
        )

    return k


def _sc_segment_max(n_entries, chunk, acc_rows, out_cols, n_pts_pad):
    nchunks = n_entries // chunk
    cpts = chunk // 4
    nbatch = chunk // 16

    @functools.partial(
        pl.kernel,
        mesh=_mesh(),
        compiler_params=pltpu.CompilerParams(
            needs_layout_passes=False, use_tc_tiling_on_sc=False),
        out_type=jax.ShapeDtypeStruct((NTILES * out_cols,), F32),
        scratch_types=[
            pltpu.VMEM((acc_rows,), F32),
            pltpu.VMEM((chunk,), I32),
            pltpu.VMEM((cpts,), F32),
        ],
    )
    def k(splat_hbm, ht_hbm, out_hbm, acc, idxb, hb):
        c = lax.axis_index("c")
        s = lax.axis_index("s")
        wid = c * 16 + s
        hbase = wid * n_pts_pad
        obase = wid * out_cols
        lanes = lax.iota(I32, 16)
        neg = jnp.full((16,), -jnp.inf, F32)

        def init(i, carry):
            acc[pl.ds(i * 16, 16)] = neg
            return carry

        lax.fori_loop(0, acc_rows // 16, init, 0)

        def do_chunk(ci, carry):
            pltpu.sync_copy(splat_hbm.at[pl.ds(ci * chunk, chunk)], idxb)
            pltpu.sync_copy(ht_hbm.at[pl.ds(hbase + ci * cpts, cpts)], hb)

            def batch(b, bc):
                kv = idxb[pl.ds(b * 16, 16)]
                pidx = lax.shift_right_logical(b * 16 + lanes, 2)
                vals = plsc.load_gather(hb, [pidx])
                ks, vs = plsc.sort_key_val(kv, vals)
                for sh in (1, 2, 4, 8):
                    sl = jnp.maximum(lanes - sh, 0)
                    k2 = _lane_gather(ks, sl)
                    v2 = _lane_gather(vs, sl)
                    m = (k2 == ks) & (lanes >= sh)
                    vs = jnp.where(m, jnp.maximum(vs, v2), vs)
                nxt = _lane_gather(ks, jnp.minimum(lanes + 1, 15))
                is_last = (ks != nxt) | (lanes == 15)
                cur = plsc.load_gather(acc, [ks])
                plsc.store_scatter(acc, [ks], jnp.maximum(cur, vs), mask=is_last)
                return bc

            lax.fori_loop(0, nbatch, batch, 0)
            return carry

        lax.fori_loop(0, nchunks, do_chunk, 0)

        def fin(i, carry):
            t = acc[pl.ds(i * 16, 16)]
            acc[pl.ds(i * 16, 16)] = jnp.where(t == neg, jnp.zeros((16,), F32), t)
            return carry

        lax.fori_loop(0, out_cols // 16, fin, 0)
        pltpu.sync_copy(acc.at[pl.ds(0, out_cols)], out_hbm.at[pl.ds(obase, out_cols)])

    return k


def _sc_gather_rows(D, per_tile, SB, table_rows):
    nsub = per_tile // SB
    assert nsub % 2 == 0
    n_out = NTILES * per_tile

    @functools.partial(
        pl.kernel,
        mesh=_mesh(),
        compiler_params=pltpu.CompilerParams(use_tc_tiling_on_sc=False),
        out_type=jax.ShapeDtypeStruct((n_out, D), F32),
        scratch_types=[
            pltpu.VMEM((per_tile,), I32),
            pltpu.VMEM((2, SB, D), F32),
            pltpu.SemaphoreType.DMA,
            pltpu.SemaphoreType.DMA,
        ],
    )
    def k(idx_hbm, tab_hbm, out_hbm, idxb, rows, sem0, sem1):
        c = lax.axis_index("c")
        s = lax.axis_index("s")
        wid = c * 16 + s
        base = wid * per_tile
        pltpu.sync_copy(idx_hbm.at[pl.ds(base, per_tile)], idxb)

        pltpu.async_copy(tab_hbm.at[idxb.at[pl.ds(0, SB)]], rows.at[0], sem0)

        def step(i, carry):
            j = i * 2
            pltpu.async_copy(
                tab_hbm.at[idxb.at[pl.ds((j + 1) * SB, SB)]], rows.at[1], sem1)
            pltpu.make_async_copy(
                tab_hbm.at[idxb.at[pl.ds(j * SB, SB)]], rows.at[0], sem0).wait()
            pltpu.sync_copy(rows.at[0], out_hbm.at[pl.ds(base + j * SB, SB), :])

            @pl.when(j + 2 < nsub)
            def _():
                pltpu.async_copy(
                    tab_hbm.at[idxb.at[pl.ds((j + 2) * SB, SB)]], rows.at[0], sem0)

            pltpu.make_async_copy(
                tab_hbm.at[idxb.at[pl.ds((j + 1) * SB, SB)]], rows.at[1], sem1).wait()
            pltpu.sync_copy(
                rows.at[1], out_hbm.at[pl.ds(base + (j + 1) * SB, SB), :])
            return carry

        lax.fori_loop(0, nsub // 2, step, 0)

    return k


def _sc_gather_sum4(D, per_tile, SB, table_rows):
    nsub = per_tile // SB
    n_out = NTILES * per_tile

    assert nsub % 2 == 0

    @functools.partial(
        pl.kernel,
        mesh=_mesh(),
        compiler_params=pltpu.CompilerParams(use_tc_tiling_on_sc=False),
        out_type=jax.ShapeDtypeStruct((n_out, D), F32),
        scratch_types=[
            pltpu.VMEM((4, per_tile), I32),
            pltpu.VMEM((8, SB, D), F32),
            pltpu.VMEM((SB, D), F32),
            pltpu.SemaphoreType.DMA,
            pltpu.SemaphoreType.DMA,
        ],
    )
    def k(idx_hbm, tab_hbm, out_hbm, idxb, rows, obuf, sem0, sem1):
        c = lax.axis_index("c")
        s = lax.axis_index("s")
        wid = c * 16 + s
        base = wid * per_tile
        for jj in range(4):
            pltpu.sync_copy(idx_hbm.at[jj, pl.ds(base, per_tile)], idxb.at[jj])

        def fire(j, g, sem):
            for jj in range(4):
                pltpu.async_copy(
                    tab_hbm.at[idxb.at[jj, pl.ds(j * SB, SB)]],
                    rows.at[g * 4 + jj], sem)

        def drain_sum_out(j, g, sem):
            for jj in range(4):
                pltpu.make_async_copy(
                    tab_hbm.at[idxb.at[jj, pl.ds(j * SB, SB)]],
                    rows.at[g * 4 + jj], sem).wait()

            def row(i, carry):
                for t in range(D // 16):
                    sl = pl.ds(t * 16, 16)
                    v = rows[g * 4 + 0, i, sl] + rows[g * 4 + 1, i, sl]
                    w = rows[g * 4 + 2, i, sl] + rows[g * 4 + 3, i, sl]
                    obuf[i, sl] = v + w
                return carry

            lax.fori_loop(0, SB, row, 0)
            pltpu.sync_copy(obuf, out_hbm.at[pl.ds(base + j * SB, SB), :])

        fire(0, 0, sem0)

        def pair(i, carry):
            j = i * 2
            fire(j + 1, 1, sem1)
            drain_sum_out(j, 0, sem0)

            @pl.when(j + 2 < nsub)
            def _():
                fire(j + 2, 0, sem0)

            drain_sum_out(j + 1, 1, sem1)
            return carry

        lax.fori_loop(0, nsub // 2, pair, 0)

    return k


def _rb(r, c):
    return pl.BlockSpec((r, c), lambda i: (i, 0))


def _rb3(d0, r, c):
    return pl.BlockSpec((d0, r, c), lambda i: (0, i, 0))


def _full(shape):
    nd = len(shape)
    return pl.BlockSpec(shape, lambda i: (0,) * nd)


def _dot(a, b):
    return jnp.dot(a, b, preferred_element_type=F32)


def _tc_pointnet(featT, w1t, b1, w2t, b2, w3t, b3, np_pad, bs=1024):
    def body(f, wa, ba, wb, bb, wc, bc, o):
        h = jnp.maximum(_dot(wa[...], f[...]) + ba[...], 0.0)
        h = jnp.maximum(_dot(wb[...], h) + bb[...], 0.0)
        o[...] = _dot(wc[...], h) + bc[...]

    return pl.pallas_call(
        body,
        grid=(np_pad // bs,),
        in_specs=[
            pl.BlockSpec((4, bs), lambda i: (0, i)),
            _full((8, 4)), _full((8, 1)),
            _full((16, 8)), _full((16, 1)),
            _full((32, 16)), _full((32, 1)),
        ],
        out_specs=pl.BlockSpec((32, bs), lambda i: (0, i)),
        out_shape=jax.ShapeDtypeStruct((32, np_pad), F32),
    )(featT, w1t, b1, w2t, b2, w3t, b3)


def _tc_invdeg(deg2, n_rows, bs):
    def body(d, o):
        t = d[0] + d[1]
        r = 1.0 / jnp.maximum(t, 1.0)
        o[...] = jnp.concatenate([r, r], axis=1)

    return pl.pallas_call(
        body,
        grid=(n_rows // bs,),
        in_specs=[_rb3(2, bs, 8)],
        out_specs=_rb(bs, 16),
        out_shape=jax.ShapeDtypeStruct((n_rows, 16), F32),
    )(deg2)


def _tc_stage1(lv_parts, ws, wn, b, n_rows, bs=512):
    def body(f, a, c, bb, x0, so, uo):
        m = jnp.max(f[...], axis=0)
        xv = jnp.transpose(jnp.where(m == -jnp.inf, 0.0, m))
        xr = jnp.maximum(xv, 0.0)
        x0[...] = xv
        so[...] = _dot(xr, a[...]) + bb[...]
        uo[...] = _dot(xr, c[...])

    return pl.pallas_call(
        body,
        grid=(n_rows // bs,),
        in_specs=[
            pl.BlockSpec((1, 32, bs), lambda i: (0, 0, i)),
            _full((32, 32)), _full((32, 32)), _full((1, 32)),
        ],
        out_specs=[_rb(bs, 32), _rb(bs, 32), _rb(bs, 32)],
        out_shape=[jax.ShapeDtypeStruct((n_rows, 32), F32)] * 3,
    )(lv_parts, ws, wn, b)


def _tc_combine_preconv(s_prev, agg2, invd, x_prev, ws, wn, b, n_rows, cx, bs=512):
    def body(sp, ag, iv, xp, a, c, bb, yo, so, uo):
        y = sp[...] + (ag[0] + ag[1]) * iv[..., :1]
        z = jnp.concatenate([xp[...], y], axis=1)
        zr = jnp.maximum(z, 0.0)
        yo[...] = y
        so[...] = _dot(zr, a[...]) + bb[...]
        uo[...] = _dot(zr, c[...])

    return pl.pallas_call(
        body,
        grid=(n_rows // bs,),
        in_specs=[
            _rb(bs, 32), _rb3(2, bs, 32), _rb(bs, 16), _rb(bs, cx),
            _full((cx + 32, 32)), _full((cx + 32, 32)), _full((1, 32)),
        ],
        out_specs=[_rb(bs, 32), _rb(bs, 32), _rb(bs, 32)],
        out_shape=[
            jax.ShapeDtypeStruct((n_rows, 32), F32),
            jax.ShapeDtypeStruct((n_rows, 32), F32),
            jax.ShapeDtypeStruct((n_rows, 32), F32),
        ],
    )(s_prev, agg2, invd, x_prev, ws, wn, b)


def _tc_finish_block(s_prev, agg2, invd, x_prev, y1, n_rows, cx, relu_out, bs=512):
    def body(sp, ag, iv, xp, y1r, lvo, xro):
        y2 = sp[...] + (ag[0] + ag[1]) * iv[..., :1]
        lv = jnp.concatenate([xp[...], y1r[...], y2], axis=1)
        lvo[...] = lv
        xro[...] = jnp.maximum(lv, 0.0)

    def body_norelu(sp, ag, iv, xp, y1r, lvo):
        y2 = sp[...] + (ag[0] + ag[1]) * iv[..., :1]
        lvo[...] = jnp.concatenate([xp[...], y1r[...], y2], axis=1)

    cout = cx + 64
    outs = [jax.ShapeDtypeStruct((n_rows, cout), F32)]
    out_specs = [_rb(bs, cout)]
    if relu_out:
        outs.append(jax.ShapeDtypeStruct((n_rows, cout), F32))
        out_specs.append(_rb(bs, cout))
    return pl.pallas_call(
        body if relu_out else body_norelu,
        grid=(n_rows // bs,),
        in_specs=[_rb(bs, 32), _rb3(2, bs, 32), _rb(bs, 16), _rb(bs, cx), _rb(bs, 32)],
        out_specs=out_specs,
        out_shape=outs,
    )(s_prev, agg2, invd, x_prev, y1)


def _tc_down(coar2, invd, wdown, bdown, wsa, wna, ba, n_rows, bs=448):
    def body(cg, iv, wd, bd, a, c, bb, lvo, so, uo):
        m = (cg[0] + cg[1]) * iv[..., :1]
        lv2 = _dot(m, wd[...]) + bd[...]
        xr = jnp.maximum(lv2, 0.0)
        lvo[...] = lv2
        so[...] = _dot(xr, a[...]) + bb[...]
        uo[...] = _dot(xr, c[...])

    return pl.pallas_call(
        body,
        grid=(n_rows // bs,),
        in_specs=[
            _rb3(2, bs, 96), _rb(bs, 16), _full((96, 96)), _full((1, 96)),
            _full((96, 32)), _full((96, 32)), _full((1, 32)),
        ],
        out_specs=[_rb(bs, 96), _rb(bs, 32), _rb(bs, 32)],
        out_shape=[
            jax.ShapeDtypeStruct((n_rows, 96), F32),
            jax.ShapeDtypeStruct((n_rows, 32), F32),
            jax.ShapeDtypeStruct((n_rows, 32), F32),
        ],
    )(coar2, invd, wdown, bdown, wsa, wna, ba)


def _tc_up(s_prev, agg2, invd, lv2, y3, wup, bup, n_rows, bs=448):
    def body(sp, ag, iv, l2, y3r, wu, bu, upo):
        y4 = sp[...] + (ag[0] + ag[1]) * iv[..., :1]
        lvf = jnp.concatenate([l2[...], y3r[...], y4], axis=1)
        upo[...] = _dot(jnp.maximum(lvf, 0.0), wu[...]) + bu[...]

    return pl.pallas_call(
        body,
        grid=(n_rows // bs,),
        in_specs=[
            _rb(bs, 32), _rb3(2, bs, 32), _rb(bs, 16), _rb(bs, 96), _rb(bs, 32),
            _full((160, 96)), _full((1, 96)),
        ],
        out_specs=_rb(bs, 96),
        out_shape=jax.ShapeDtypeStruct((n_rows, 96), F32),
    )(s_prev, agg2, invd, lv2, y3, wup, bup)


def _tc_skip_preconv(upg, lv96, wsa, wna, ba, n_rows, bs=512):
    def body(ug, l9, a, c, bb, lvo, so, uo):
        lv = ug[...] + l9[...]
        xr = jnp.maximum(lv, 0.0)
        lvo[...] = lv
        so[...] = _dot(xr, a[...]) + bb[...]
        uo[...] = _dot(xr, c[...])

    return pl.pallas_call(
        body,
        grid=(n_rows // bs,),
        in_specs=[
            _rb(bs, 96), _rb(bs, 96),
            _full((96, 32)), _full((96, 32)), _full((1, 32)),
        ],
        out_specs=[_rb(bs, 96), _rb(bs, 32), _rb(bs, 32)],
        out_shape=[
            jax.ShapeDtypeStruct((n_rows, 96), F32),
            jax.ShapeDtypeStruct((n_rows, 32), F32),
            jax.ShapeDtypeStruct((n_rows, 32), F32),
        ],
    )(upg, lv96, wsa, wna, ba)


def _tc_head(gsum, wsl, bsl, w1, b1, w2, b2, n_rows, bs=512):
    def body(gh, a, ba, bmat, bb, cmat, bc, o):
        g = gh[...] * 0.25
        sv = _dot(jnp.maximum(g, 0.0), a[...]) + ba[...]
        h2 = jnp.maximum(_dot(jnp.maximum(sv, 0.0), bmat[...]) + bb[...], 0.0)
        lg = _dot(h2, cmat[...]) + bc[...]
        mx = jnp.max(lg, axis=1, keepdims=True)
        e = jnp.exp(lg - mx)
        o[...] = lg - mx - jnp.log(jnp.sum(e, axis=1, keepdims=True))

    return pl.pallas_call(
        body,
        grid=(n_rows // bs,),
        in_specs=[
            _rb(bs, 160),
            _full((160, 64)), _full((1, 64)),
            _full((64, 32)), _full((1, 32)),
            _full((32, 20)), _full((1, 20)),
        ],
        out_specs=_rb(bs, 20),
        out_shape=jax.ShapeDtypeStruct((n_rows, 20), F32),
    )(gsum, wsl, bsl, w1, b1, w2, b2)


def kernel(positions, values, params, splat_idx, edge_index, coarse_idx, coarse_edge_index):
    p = params
    n = positions.shape[0]
    v = 25000
    e = edge_index.shape[1]
    ec = coarse_edge_index.shape[1]

    np_pad = 51200
    vp = 25088
    vacc = 25216
    vdum = 25100
    vcp = 6272
    vcacc = 6400
    vcdum = 6300

    i32 = lambda x: x.astype(I32)

    ent = 4 * n
    ent_pad = 204800
    splat_flat = i32(splat_idx.reshape(-1))
    splat_max = jnp.concatenate(
        [splat_flat, jnp.full((ent_pad - ent,), vdum, I32)])
    splat_T = jnp.concatenate(
        [i32(splat_idx), jnp.zeros((np_pad - n, 4), I32)]).T

    e_pad = 409600
    src_f = jnp.concatenate([i32(edge_index[0]), jnp.zeros((e_pad - e,), I32)])
    dst_f = jnp.concatenate(
        [i32(edge_index[1]), jnp.full((e_pad - e,), vdum, I32)]
    ).reshape(NTILES, 100, 128)

    ec_pad = 102400
    src_c = jnp.concatenate([i32(coarse_edge_index[0]), jnp.zeros((ec_pad - ec,), I32)])
    dst_c = jnp.concatenate(
        [i32(coarse_edge_index[1]), jnp.full((ec_pad - ec,), vcdum, I32)]
    ).reshape(NTILES, 20, 160)

    v_pad = 25600
    src_s = jnp.concatenate([jnp.arange(v, dtype=I32), jnp.zeros((v_pad - v,), I32)])
    dst_s = jnp.concatenate(
        [i32(coarse_idx), jnp.full((v_pad - v,), vcdum, I32)]
    ).reshape(NTILES, 10, 80)
    cidx_gather = jnp.concatenate([i32(coarse_idx), jnp.zeros((v_pad - v,), I32)])

    seg32_fine = _sc_segment_sum(32, 12800, 128, vacc, vp)
    seg32_coarse = _sc_segment_sum(32, 3200, 160, vcacc, vcp)
    seg96_down = _sc_segment_sum(96, 800, 80, vcacc, vp)
    deg_fine_k = _sc_degree(12800, 128, vacc)
    deg_coarse_k = _sc_degree(3200, 160, vcacc)
    deg_down_k = _sc_degree(800, 80, vcacc)
    segmax_k = _sc_segment_max(ent_pad, 1600, vacc, vp, np_pad)
    gather96 = _sc_gather_rows(96, 800, 80, vcp)
    gathersum160 = _sc_gather_sum4(160, 1600, 40, vp)

    z_f = jnp.zeros((vacc // 16, 8), F32)
    z_c = jnp.zeros((vcacc // 16, 8), F32)
    zf32 = jnp.zeros((vacc, 32), F32)
    zc32 = jnp.zeros((vcacc, 32), F32)
    zc96 = jnp.zeros((vcacc, 96), F32)
    invd_f = _tc_invdeg(deg_fine_k(dst_f, jnp.ones((128, 8), F32), z_f), vp, 512)
    invd_c = _tc_invdeg(deg_coarse_k(dst_c, jnp.ones((160, 8), F32), z_c), vcp, 448)
    invd_s = _tc_invdeg(deg_down_k(dst_s, jnp.ones((80, 8), F32), z_c), vcp, 448)

    feat = jnp.concatenate([positions, values], axis=1)
    featT = jnp.concatenate(
        [feat, jnp.zeros((np_pad - n, 4), F32)]).T
    hT = _tc_pointnet(
        featT,
        p['W_p1'].T, p['b_p1'].reshape(8, 1),
        p['W_p2'].T, p['b_p2'].reshape(16, 1),
        p['W_p3'].T, p['b_p3'].reshape(32, 1),
        np_pad)
    lv0_parts = segmax_k(splat_max, hT.reshape(-1)).reshape(1, 32, vp)

    x0, s1, u1 = _tc_stage1(
        lv0_parts, p['d1a_s'], p['d1a_n'], p['d1a_b'].reshape(1, 32), vp)
    agg1 = seg32_fine(src_f, dst_f, u1, zf32)
    y1, s2, u2 = _tc_combine_preconv(
        s1, agg1, invd_f, x0, p['d1b_s'], p['d1b_n'], p['d1b_b'].reshape(1, 32), vp, 32)
    agg2 = seg32_fine(src_f, dst_f, u2, zf32)
    lv96, xr96 = _tc_finish_block(s2, agg2, invd_f, x0, y1, vp, 32, True)

    coar = seg96_down(src_s, dst_s, xr96, zc96)
    lv2, s3, u3 = _tc_down(
        coar, invd_s, p['W_down'], p['b_down'].reshape(1, 96),
        p['d2a_s'], p['d2a_n'], p['d2a_b'].reshape(1, 32), vcp)
    agg3 = seg32_coarse(src_c, dst_c, u3, zc32)
    y3, s4, u4 = _tc_combine_preconv(
        s3, agg3, invd_c, lv2, p['d2b_s'], p['d2b_n'], p['d2b_b'].reshape(1, 32), vcp, 96, bs=448)
    agg4 = seg32_coarse(src_c, dst_c, u4, zc32)
    up = _tc_up(s4, agg4, invd_c, lv2, y3, p['W_up'], p['b_up'].reshape(1, 96), vcp)

    upg = gather96(cidx_gather, up)
    lv96b, s5, u5 = _tc_skip_preconv(
        upg[:vp], lv96, p['d3a_s'], p['d3a_n'], p['d3a_b'].reshape(1, 32), vp)
    agg5 = seg32_fine(src_f, dst_f, u5, zf32)
    y5, s6, u6 = _tc_combine_preconv(
        s5, agg5, invd_f, lv96b, p['d3b_s'], p['d3b_n'], p['d3b_b'].reshape(1, 32), vp, 96)
    agg6 = seg32_fine(src_f, dst_f, u6, zf32)
    (lvfin,) = _tc_finish_block(s6, agg6, invd_f, lv96b, y5, vp, 96, False)

    gsum = gathersum160(splat_T, lvfin)
    out = _tc_head(
        gsum, p['W_sl'], p['b_sl'].reshape(1, 64),
        p['W_s1'], p['b_s1'].reshape(1, 32),
        p['W_s2'], p['b_s2'].reshape(1, 20), np_pad)
    return out[:n].reshape(1, n, 20)
